# baseline (speedup 1.0000x reference)
import functools
import math

import numpy as np
import jax
import jax.numpy as jnp
from jax import lax
from jax.experimental import pallas as pl
from jax.experimental.pallas import tpu as pltpu
from jax.experimental.pallas import tpu_sc as plsc

_VSZ = 1000000
_DSZ = 64
_B = 1024
_T = 200
_ROWS = _B * _T
_NW = 32
_PER_W = _ROWS // _NW
_CHUNK = 128
_NCHUNK = _PER_W // _CHUNK
_SCALE = math.sqrt(_DSZ)
_PEROWS = 320


def _build_pe_pairs() -> np.ndarray:
    log_timescale_increment = math.log(10000.0) / float(_DSZ)
    inv_timescales = np.exp(
        np.arange(0, _DSZ, 2, dtype=np.float32) * -log_timescale_increment)
    pe = np.zeros((_T, _DSZ), dtype=np.float32)
    position = np.arange(0, _T, dtype=np.float32)[:, None]
    pe[:, 0::2] = np.sin(position * inv_timescales)
    pe[:, 1::2] = np.cos(position * inv_timescales)
    big = pe[np.arange(_PEROWS) % _T]
    return big.reshape(_PEROWS // 2, 2 * _DSZ)


_PE_PAIRS = _build_pe_pairs()

_mesh = plsc.VectorSubcoreMesh(core_axis_name="c", subcore_axis_name="s")


@functools.partial(
    pl.kernel,
    mesh=_mesh,
    compiler_params=pltpu.CompilerParams(use_tc_tiling_on_sc=True),
    out_type=jax.ShapeDtypeStruct((_ROWS // 2, 2 * _DSZ), jnp.float32),
    scratch_types=[
        pltpu.VMEM((_NCHUNK, 128), jnp.int32),
        pltpu.VMEM((_NCHUNK, 128), jnp.int32),
        pltpu.VMEM((_CHUNK, 2 * _DSZ), jnp.float32),
        pltpu.VMEM((_CHUNK // 2, 2 * _DSZ), jnp.float32),
        pltpu.VMEM((_PEROWS // 2, 2 * _DSZ), jnp.float32),
        pltpu.SemaphoreType.DMA,
    ],
)
def _sc_embed(w_hbm, idx_hbm, pe_hbm, out_hbm, idx_v, idxp_v, rows_v, out_v,
              pe_v, sem):
    wid = lax.axis_index("s") * 2 + lax.axis_index("c")
    pltpu.sync_copy(pe_hbm, pe_v)
    pltpu.sync_copy(idx_hbm.at[wid], idx_v)

    def pair_body(g, carry):
        row = g // 8
        sl = pl.ds((g % 8) * 16, 16)
        idxp_v[row, sl] = lax.shift_right_logical(idx_v[row, sl], 1)
        return carry

    lax.fori_loop(0, _NCHUNK * 8, pair_body, 0)

    def chunk_body(c, carry):
        base = wid * _PER_W + c * _CHUNK
        off0 = base % _T
        pltpu.async_copy(w_hbm.at[idxp_v.at[c]], rows_v, sem).wait()

        def grp_body(g, carry2):
            iv = idx_v[c, pl.ds(g * 16, 16)]
            sv = jnp.where(iv == 0, 0.0, _SCALE).astype(jnp.float32)
            hv = (iv & 1) * _DSZ
            for rl in range(16):
                r = g * 16 + rl
                s_r = sv[rl]
                h_r = hv[rl]
                half = (rl % 2) * _DSZ
                pj = (off0 + r) // 2
                for q in range(_DSZ // 16):
                    out_v[r // 2, pl.ds(half + q * 16, 16)] = (
                        rows_v[r, pl.ds(h_r + q * 16, 16)] * s_r
                        + pe_v[pj, pl.ds(half + q * 16, 16)]
                    )
            return carry2

        lax.fori_loop(0, _CHUNK // 16, grp_body, 0)
        obase = pl.multiple_of(base // 2, _CHUNK // 2)
        pltpu.sync_copy(out_v, out_hbm.at[pl.ds(obase, _CHUNK // 2)])
        return carry

    lax.fori_loop(0, _NCHUNK, chunk_body, 0)


def kernel(x, W):
    B, T = x.shape
    assert (B, T) == (_B, _T) and W.shape == (_VSZ, _DSZ)
    xf = x.reshape(_NW, _NCHUNK, 128).astype(jnp.int32)
    w2 = W.astype(jnp.float32).reshape(_VSZ // 2, 2 * _DSZ)
    out = _sc_embed(w2, xf, jnp.asarray(_PE_PAIRS))
    return out.reshape(B, T, _DSZ)


# speedup vs baseline: 1.0973x; 1.0973x over previous
import functools
import math

import numpy as np
import jax
import jax.numpy as jnp
from jax import lax
from jax.experimental import pallas as pl
from jax.experimental.pallas import tpu as pltpu
from jax.experimental.pallas import tpu_sc as plsc

_VSZ = 1000000
_DSZ = 64
_B = 1024
_T = 200
_ROWS = _B * _T
_NW = 32
_PER_W = _ROWS // _NW
_CHUNK = 128
_NCHUNK = _PER_W // _CHUNK
_SCALE = math.sqrt(_DSZ)
_PEROWS = 320


def _build_pe_pairs() -> np.ndarray:
    log_timescale_increment = math.log(10000.0) / float(_DSZ)
    inv_timescales = np.exp(
        np.arange(0, _DSZ, 2, dtype=np.float32) * -log_timescale_increment)
    pe = np.zeros((_T, _DSZ), dtype=np.float32)
    position = np.arange(0, _T, dtype=np.float32)[:, None]
    pe[:, 0::2] = np.sin(position * inv_timescales)
    pe[:, 1::2] = np.cos(position * inv_timescales)
    big = pe[np.arange(_PEROWS) % _T]
    return big.reshape(_PEROWS // 2, 2 * _DSZ)


_PE_PAIRS = _build_pe_pairs()

_mesh = plsc.VectorSubcoreMesh(core_axis_name="c", subcore_axis_name="s")


@functools.partial(
    pl.kernel,
    mesh=_mesh,
    compiler_params=pltpu.CompilerParams(use_tc_tiling_on_sc=True),
    out_type=jax.ShapeDtypeStruct((_ROWS // 2, 2 * _DSZ), jnp.float32),
    scratch_types=[
        pltpu.VMEM((_NCHUNK, 128), jnp.int32),
        pltpu.VMEM((_NCHUNK, 128), jnp.int32),
        pltpu.VMEM((_CHUNK, 128), jnp.float32),
        pltpu.VMEM((_CHUNK, 128), jnp.float32),
        pltpu.VMEM((_CHUNK // 2, 2 * _DSZ), jnp.float32),
        pltpu.VMEM((_CHUNK // 2, 2 * _DSZ), jnp.float32),
        pltpu.VMEM((_PEROWS // 2, 2 * _DSZ), jnp.float32),
        pltpu.SemaphoreType.DMA,
        pltpu.SemaphoreType.DMA,
        pltpu.SemaphoreType.DMA,
        pltpu.SemaphoreType.DMA,
    ],
)
def _sc_embed(w_hbm, idx_hbm, idxp_hbm, pe_hbm, out_hbm, idx_v, idxp_v,
              rows_a, rows_b, out_a, out_b, pe_v, sem_ga, sem_gb, sem_wa,
              sem_wb):
    wid = lax.axis_index("s") * 2 + lax.axis_index("c")
    pltpu.sync_copy(pe_hbm, pe_v)
    pltpu.sync_copy(idx_hbm.at[wid], idx_v)
    pltpu.sync_copy(idxp_hbm.at[wid], idxp_v)

    def gstart(c, slot, buf, sem):
        pltpu.async_copy(w_hbm.at[idxp_v.at[c]], buf, sem)

    def gwait(buf, sem):
        pltpu.make_async_copy(w_hbm.at[idxp_v.at[0]], buf, sem).wait()

    def wstart(c, buf, sem):
        base2 = pl.multiple_of((wid * _PER_W + c * _CHUNK) // 2, _CHUNK // 2)
        pltpu.async_copy(buf, out_hbm.at[pl.ds(base2, _CHUNK // 2)], sem)

    def wwait(buf, sem):
        pltpu.make_async_copy(buf, out_hbm.at[pl.ds(0, _CHUNK // 2)], sem).wait()

    def compute(c, rows_v, out_v):
        base = wid * _PER_W + c * _CHUNK
        off0 = base % _T

        def grp_body(g, carry):
            iv = idx_v[c, pl.ds(g * 16, 16)]
            sv = jnp.where(iv == 0, 0.0, _SCALE).astype(jnp.float32)
            hv = (iv & 1) * _DSZ
            for rl in range(16):
                r = g * 16 + rl
                s_r = sv[rl]
                h_r = hv[rl]
                half = (rl % 2) * _DSZ
                pj = (off0 + r) // 2
                for q in range(_DSZ // 16):
                    out_v[r // 2, pl.ds(half + q * 16, 16)] = (
                        rows_v[r, pl.ds(h_r + q * 16, 16)] * s_r
                        + pe_v[pj, pl.ds(half + q * 16, 16)]
                    )
            return carry

        lax.fori_loop(0, _CHUNK // 16, grp_body, 0)

    gstart(0, 0, rows_a, sem_ga)

    def body(k, carry):
        c0 = 2 * k
        c1 = 2 * k + 1
        gstart(c1, 1, rows_b, sem_gb)
        gwait(rows_a, sem_ga)

        @pl.when(k > 0)
        def _():
            wwait(out_a, sem_wa)

        compute(c0, rows_a, out_a)
        wstart(c0, out_a, sem_wa)

        @pl.when(k < _NCHUNK // 2 - 1)
        def _():
            gstart(c0 + 2, 0, rows_a, sem_ga)

        gwait(rows_b, sem_gb)

        @pl.when(k > 0)
        def _():
            wwait(out_b, sem_wb)

        compute(c1, rows_b, out_b)
        wstart(c1, out_b, sem_wb)
        return carry

    lax.fori_loop(0, _NCHUNK // 2, body, 0)
    wwait(out_a, sem_wa)
    wwait(out_b, sem_wb)


def kernel(x, W):
    B, T = x.shape
    assert (B, T) == (_B, _T) and W.shape == (_VSZ, _DSZ)
    xf = x.reshape(_NW, _NCHUNK, 128).astype(jnp.int32)
    xp = lax.shift_right_logical(xf, 1)
    w2 = W.astype(jnp.float32).reshape(_VSZ // 2, 2 * _DSZ)
    out = _sc_embed(w2, xf, xp, jnp.asarray(_PE_PAIRS))
    return out.reshape(B, T, _DSZ)


# speedup vs baseline: 1.1990x; 1.0926x over previous
import functools
import math

import numpy as np
import jax
import jax.numpy as jnp
from jax import lax
from jax.experimental import pallas as pl
from jax.experimental.pallas import tpu as pltpu
from jax.experimental.pallas import tpu_sc as plsc

_VSZ = 1000000
_DSZ = 64
_B = 1024
_T = 200
_ROWS = _B * _T
_NW = 32
_PER_W = _ROWS // _NW
_CHUNK = 128
_NCHUNK = _PER_W // _CHUNK
_SCALE = math.sqrt(_DSZ)
_PEROWS = 320


def _build_pe_pairs() -> np.ndarray:
    log_timescale_increment = math.log(10000.0) / float(_DSZ)
    inv_timescales = np.exp(
        np.arange(0, _DSZ, 2, dtype=np.float32) * -log_timescale_increment)
    pe = np.zeros((_T, _DSZ), dtype=np.float32)
    position = np.arange(0, _T, dtype=np.float32)[:, None]
    pe[:, 0::2] = np.sin(position * inv_timescales)
    pe[:, 1::2] = np.cos(position * inv_timescales)
    big = pe[np.arange(_PEROWS) % _T]
    return big.reshape(_PEROWS // 2, 2 * _DSZ)


_PE_PAIRS = _build_pe_pairs()

_mesh = plsc.VectorSubcoreMesh(core_axis_name="c", subcore_axis_name="s")


@functools.partial(
    pl.kernel,
    mesh=_mesh,
    compiler_params=pltpu.CompilerParams(use_tc_tiling_on_sc=True),
    out_type=jax.ShapeDtypeStruct((_ROWS, _DSZ), jnp.float32),
    scratch_types=[
        pltpu.VMEM((_NCHUNK, 128), jnp.int32),
        pltpu.VMEM((_NCHUNK, 128), jnp.int32),
        pltpu.VMEM((_CHUNK, 128), jnp.float32),
        pltpu.VMEM((_CHUNK, 128), jnp.float32),
        pltpu.VMEM((_CHUNK, _DSZ), jnp.float32),
        pltpu.VMEM((_CHUNK, _DSZ), jnp.float32),
        pltpu.VMEM((_PEROWS // 2, 2 * _DSZ), jnp.float32),
        pltpu.SemaphoreType.DMA,
        pltpu.SemaphoreType.DMA,
        pltpu.SemaphoreType.DMA,
        pltpu.SemaphoreType.DMA,
    ],
)
def _sc_embed(w_hbm, idx_hbm, idxp_hbm, pe_hbm, out_hbm, idx_v, idxp_v,
              rows_a, rows_b, out_a, out_b, pe_v, sem_ga, sem_gb, sem_wa,
              sem_wb):
    wid = lax.axis_index("s") * 2 + lax.axis_index("c")
    pltpu.sync_copy(pe_hbm, pe_v)
    pltpu.sync_copy(idx_hbm.at[wid], idx_v)
    pltpu.sync_copy(idxp_hbm.at[wid], idxp_v)

    def gstart(c, slot, buf, sem):
        pltpu.async_copy(w_hbm.at[idxp_v.at[c]], buf, sem)

    def gwait(buf, sem):
        pltpu.make_async_copy(w_hbm.at[idxp_v.at[0]], buf, sem).wait()

    def wstart(c, buf, sem):
        base = pl.multiple_of(wid * _PER_W + c * _CHUNK, _CHUNK)
        pltpu.async_copy(buf, out_hbm.at[pl.ds(base, _CHUNK)], sem)

    def wwait(buf, sem):
        pltpu.make_async_copy(buf, out_hbm.at[pl.ds(0, _CHUNK)], sem).wait()

    def compute(c, rows_v, out_v):
        base = wid * _PER_W + c * _CHUNK
        off0 = base % _T

        def grp_body(g, carry):
            iv = idx_v[c, pl.ds(g * 16, 16)]
            sv = jnp.where(iv == 0, 0.0, _SCALE).astype(jnp.float32)
            hv = (iv & 1) * _DSZ
            for rl in range(16):
                r = g * 16 + rl
                s_r = sv[rl]
                h_r = hv[rl]
                half = (rl % 2) * _DSZ
                pj = (off0 + r) // 2
                for q in range(_DSZ // 16):
                    out_v[r, pl.ds(q * 16, 16)] = (
                        rows_v[r, pl.ds(h_r + q * 16, 16)] * s_r
                        + pe_v[pj, pl.ds(half + q * 16, 16)]
                    )
            return carry

        lax.fori_loop(0, _CHUNK // 16, grp_body, 0)

    gstart(0, 0, rows_a, sem_ga)

    def body(k, carry):
        c0 = 2 * k
        c1 = 2 * k + 1
        gstart(c1, 1, rows_b, sem_gb)
        gwait(rows_a, sem_ga)

        @pl.when(k > 0)
        def _():
            wwait(out_a, sem_wa)

        compute(c0, rows_a, out_a)
        wstart(c0, out_a, sem_wa)

        @pl.when(k < _NCHUNK // 2 - 1)
        def _():
            gstart(c0 + 2, 0, rows_a, sem_ga)

        gwait(rows_b, sem_gb)

        @pl.when(k > 0)
        def _():
            wwait(out_b, sem_wb)

        compute(c1, rows_b, out_b)
        wstart(c1, out_b, sem_wb)
        return carry

    lax.fori_loop(0, _NCHUNK // 2, body, 0)
    wwait(out_a, sem_wa)
    wwait(out_b, sem_wb)


def kernel(x, W):
    B, T = x.shape
    assert (B, T) == (_B, _T) and W.shape == (_VSZ, _DSZ)
    xf = x.reshape(_NW, _NCHUNK, 128).astype(jnp.int32)
    xp = lax.shift_right_logical(xf, 1)
    w2 = W.astype(jnp.float32).reshape(_VSZ // 2, 2 * _DSZ)
    out = _sc_embed(w2, xf, xp, jnp.asarray(_PE_PAIRS))
    return out.reshape(B, T, _DSZ)
